# Initial kernel scaffold; baseline (speedup 1.0000x reference)
import jax
import jax.numpy as jnp
from jax.experimental import pallas as pl


def kernel(target_series, TS_database):
    raise NotImplementedError("write your pallas kernel here")


# speedup vs baseline: 2.4303x; 2.4303x over previous
import functools

import numpy as np
import jax
import jax.numpy as jnp
from jax import lax
from jax.experimental import pallas as pl
from jax.experimental.pallas import tpu as pltpu
from jax.experimental.pallas import tpu_sc as plsc

NPERSEG = 128
NREF = 64
STEP = 64
NSEG = 7
F = 65
B = 8
N = 4096
L = 512
NB = 2048

_HP = lax.Precision.HIGHEST


def _dft_consts():
    k = np.arange(NPERSEG)
    f = np.arange(F)
    ang = -2.0 * np.pi * np.outer(k, f) / NPERSEG
    return np.cos(ang).astype(np.float32), np.sin(ang).astype(np.float32)


_WR_NP, _WI_NP = _dft_consts()


def _scores_body(tgt_ref, db_ref, win_ref, wr_ref, wi_ref, out_ref):
    win = win_ref[0, :]
    wr = wr_ref[...]
    wi = wi_ref[...]
    dn = (((1,), (0,)), ((), ()))
    dbb = db_ref[0]
    b = pl.program_id(0)
    tg = tgt_ref[pl.ds(b, 1), :]

    Xr, Xi = [], []
    pxx = None
    for s in range(NSEG):
        ts = tg[:, s * STEP:s * STEP + NPERSEG] * win
        xr = lax.dot_general(ts, wr, dn, precision=_HP,
                             preferred_element_type=jnp.float32)
        xi = lax.dot_general(ts, wi, dn, precision=_HP,
                             preferred_element_type=jnp.float32)
        Xr.append(xr)
        Xi.append(xi)
        t = xr * xr + xi * xi
        pxx = t if pxx is None else pxx + t
    pxx = pxx / np.float32(NSEG)

    syy = sxyr = sxyi = None
    for s in range(NSEG):
        xs = dbb[:, s * STEP:s * STEP + NPERSEG] * win
        yr = lax.dot_general(xs, wr, dn, precision=_HP,
                             preferred_element_type=jnp.float32)
        yi = lax.dot_general(xs, wi, dn, precision=_HP,
                             preferred_element_type=jnp.float32)
        t0 = yr * yr + yi * yi
        t1 = Xr[s] * yr + Xi[s] * yi
        t2 = Xi[s] * yr - Xr[s] * yi
        if syy is None:
            syy, sxyr, sxyi = t0, t1, t2
        else:
            syy, sxyr, sxyi = syy + t0, sxyr + t1, sxyi + t2
    pyy = syy / np.float32(NSEG)
    pxyr = sxyr / np.float32(NSEG)
    pxyi = sxyi / np.float32(NSEG)
    cxy = (pxyr * pxyr + pxyi * pxyi) / (pxx * pyy + np.float32(1e-12))
    out_ref[0, 0, :] = jnp.sum(cxy, axis=1) / np.float32(F)


def _topk_body(scores_ref, out_ref):
    s = scores_ref[:, 0, :]
    lane = lax.broadcasted_iota(jnp.int32, (B, N), 1)
    row_base = lax.broadcasted_iota(jnp.int32, (B, 1), 0) * N
    kiota = lax.broadcasted_iota(jnp.int32, (B, NREF), 1)

    def step(k, carry):
        s, acc = carry
        m = jnp.max(s, axis=1, keepdims=True)
        hit = s == m
        idx = jnp.min(jnp.where(hit, lane, jnp.int32(N)), axis=1,
                      keepdims=True)
        acc = jnp.where(kiota == k, idx + row_base, acc)
        s = jnp.where(lane == idx, jnp.float32(-1.0), s)
        return s, acc

    _, acc = lax.fori_loop(0, NREF, step,
                           (s, jnp.zeros((B, NREF), jnp.int32)))
    out_ref[...] = acc


def _tc_scores(tgt, db, win, wr, wi):
    return pl.pallas_call(
        _scores_body,
        grid=(B, N // NB),
        in_specs=[
            pl.BlockSpec((B, L), lambda b, n: (0, 0)),
            pl.BlockSpec((1, NB, L), lambda b, n: (b, n, 0)),
            pl.BlockSpec((1, NPERSEG), lambda b, n: (0, 0)),
            pl.BlockSpec((NPERSEG, F), lambda b, n: (0, 0)),
            pl.BlockSpec((NPERSEG, F), lambda b, n: (0, 0)),
        ],
        out_specs=pl.BlockSpec((1, 1, NB), lambda b, n: (b, 0, n)),
        out_shape=jax.ShapeDtypeStruct((B, 1, N), jnp.float32),
    )(tgt, db, win, wr, wi)


def _tc_topk(scores):
    return pl.pallas_call(
        _topk_body,
        out_shape=jax.ShapeDtypeStruct((B, NREF), jnp.int32),
    )(scores)


def _sc_gather(db_flat, idx_flat):
    info = plsc.get_sparse_core_info()
    nw = info.num_cores * info.num_subcores
    rows = B * NREF
    per_w = rows // nw
    mesh = plsc.VectorSubcoreMesh(core_axis_name="c", subcore_axis_name="s")

    @functools.partial(
        pl.kernel,
        mesh=mesh,
        out_type=jax.ShapeDtypeStruct((rows, L), jnp.float32),
        scratch_types=[
            pltpu.VMEM((per_w,), jnp.int32),
            pltpu.VMEM((per_w, L), jnp.float32),
            pltpu.SemaphoreType.DMA,
        ],
    )
    def k(db_hbm, idx_hbm, out_hbm, idx_v, rows_v, sem):
        wid = lax.axis_index("s") * info.num_cores + lax.axis_index("c")
        base = wid * per_w
        pltpu.sync_copy(idx_hbm.at[pl.ds(base, per_w)], idx_v)
        pltpu.async_copy(db_hbm.at[idx_v], rows_v, sem).wait()
        pltpu.sync_copy(rows_v, out_hbm.at[pl.ds(base, per_w)])

    return k(db_flat, idx_flat)


def kernel(target_series, TS_database):
    tgt = jnp.squeeze(target_series, axis=1)
    win = (0.5 - 0.5 * jnp.cos(
        2.0 * jnp.pi * jnp.arange(NPERSEG, dtype=jnp.float32) / NPERSEG
    ))[None, :]
    T = jnp.fft.rfft(jnp.eye(NPERSEG, dtype=jnp.float32), axis=-1)
    wr = jnp.real(T).astype(jnp.float32)
    wi = jnp.imag(T).astype(jnp.float32)
    scores = _tc_scores(tgt, TS_database, win, wr, wi)
    flat_idx = _tc_topk(scores)
    out = _sc_gather(TS_database.reshape(B * N, L),
                     flat_idx.reshape(B * NREF))
    return out.reshape(B, NREF, L)
